# baseline (speedup 1.0000x reference)
import functools

import jax
import jax.numpy as jnp
from jax import lax
from jax.experimental import pallas as pl
from jax.experimental.pallas import tpu as pltpu
from jax.experimental.pallas import tpu_sc as plsc

NUM_USERS = 25000
NUM_ITEMS = 25000
N_NODES = NUM_USERS + NUM_ITEMS
EMB_DIM = 64
N_EDGES = 800000
N_LAYERS = 3

HALF = N_NODES // 2
CH = 80
SUP = 5
SCH = SUP * CH
NCHUNKS = N_EDGES // CH
NSUB = 16
NSUPER = NCHUNKS // (NSUB * SUP)
ROWBLK = 40
NROWBLK = HALF // ROWBLK


def _layer_kernel():
    mesh = plsc.VectorSubcoreMesh(core_axis_name="c", subcore_axis_name="s",
                                  num_cores=2, num_subcores=NSUB)

    @functools.partial(
        pl.kernel,
        out_type=jax.ShapeDtypeStruct((N_NODES, EMB_DIM), jnp.float32),
        mesh=mesh,
        compiler_params=pltpu.CompilerParams(use_tc_tiling_on_sc=False),
        scratch_types=[
            pltpu.VMEM((2, 2, SCH), jnp.int32),
            pltpu.VMEM((2, SCH), jnp.float32),
            pltpu.VMEM((2, SUP, CH), jnp.int32),
            pltpu.VMEM((2, SUP, CH), jnp.float32),
            pltpu.VMEM((2, CH, EMB_DIM), jnp.float32),
            pltpu.VMEM((2, CH, EMB_DIM), jnp.float32),
            pltpu.VMEM((ROWBLK, EMB_DIM), jnp.float32),
            pltpu.VMEM_SHARED((HALF, EMB_DIM), jnp.float32),
            pltpu.SemaphoreType.DMA,
            pltpu.SemaphoreType.DMA,
            pltpu.SemaphoreType.DMA,
            pltpu.SemaphoreType.DMA,
            pltpu.SemaphoreType.DMA,
            pltpu.SemaphoreType.DMA,
        ],
    )
    def layer(table_hbm, edata_hbm, wdata_hbm, out_hbm,
              edv, wv, dl, wb, rows, rows2, zbuf, acc,
              se0, se1, sg0, sg1, ss0, ss1):
        c = lax.axis_index("c")
        s = lax.axis_index("s")
        chalf = c * HALF
        sem_e = (se0, se1)
        sem_g = (sg0, sg1)
        sem_s = (ss0, ss1)

        def sup_id(i):
            return jnp.minimum(i, NSUPER - 1)

        def load_edata(i, B):
            pltpu.async_copy(edata_hbm.at[s, sup_id(i)], edv.at[B], sem_e[B])
            pltpu.async_copy(wdata_hbm.at[s, sup_id(i)], wv.at[B], sem_e[B])

        def wait_edata(i, B):
            pltpu.make_async_copy(edata_hbm.at[s, sup_id(i)], edv.at[B],
                                  sem_e[B]).wait()
            pltpu.make_async_copy(wdata_hbm.at[s, sup_id(i)], wv.at[B],
                                  sem_e[B]).wait()

        def issue_gather(B, j, p):
            pltpu.async_copy(table_hbm.at[edv.at[B, 0, pl.ds(j * CH, CH)]],
                             rows.at[p], sem_g[p])

        def wait_gather(B, j, p):
            pltpu.make_async_copy(table_hbm.at[edv.at[B, 0, pl.ds(j * CH, CH)]],
                                  rows.at[p], sem_g[p]).wait()

        def issue_scatter(B, j, p):
            pltpu.async_copy(rows2.at[p], acc.at[dl.at[B, j]], sem_s[p],
                             add=True)

        def wait_scatter(B, j, p):
            pltpu.make_async_copy(rows2.at[p], acc.at[dl.at[B, j]],
                                  sem_s[p]).wait()

        def dfold(B):
            for v in range(SCH // 16):
                j, g = v // (CH // 16), v % (CH // 16)
                sl = pl.ds(v * 16, 16)
                d = edv[B, 1, sl]
                w = wv[B, sl]
                fold = jnp.where(d >= HALF, d - HALF, d)
                valid = (d >= chalf) & (d < chalf + HALF)
                dl[B, j, pl.ds(g * 16, 16)] = fold
                wb[B, j, pl.ds(g * 16, 16)] = jnp.where(valid, w, 0.0)

        def scale(B, j, p):
            for g in range(CH // 16):
                ev = wb[B, j, pl.ds(g * 16, 16)]
                for l in range(16):
                    e = g * 16 + l
                    wbc = jnp.broadcast_to(
                        lax.squeeze(lax.slice(ev, (l,), (l + 1,)), (0,)),
                        (16,))
                    for q in range(EMB_DIM // 16):
                        qs = pl.ds(q * 16, 16)
                        rows2[p, e, qs] = rows[p, e, qs] * wbc

        load_edata(0, 0)
        wait_edata(0, 0)
        issue_gather(0, 0, 0)
        load_edata(1, 1)

        zeros16 = jnp.zeros((16,), jnp.float32)

        def zb(t, carry):
            r = t // (EMB_DIM // 16)
            k = t % (EMB_DIM // 16)
            zbuf[r, pl.ds(k * 16, 16)] = zeros16
            return carry

        lax.fori_loop(0, ROWBLK * (EMB_DIM // 16), zb, 0)

        def zero_chunk(i, carry):
            j = s + NSUB * i
            base = pl.multiple_of(j * ROWBLK, 8)
            pltpu.sync_copy(zbuf, acc.at[pl.ds(base, ROWBLK)])
            return carry

        nz = (NROWBLK - s + NSUB - 1) // NSUB
        lax.fori_loop(0, nz, zero_chunk, 0)
        plsc.subcore_barrier()

        def body(i, B, first=False):
            dfold(B)
            for j in range(SUP):
                p = (B + j) % 2
                wait_gather(B, j, p)
                if not (first and j < 2):
                    wait_scatter(B, j, p)
                scale(B, j, p)
                if j < SUP - 1:
                    issue_gather(B, j + 1, 1 - p)
                else:
                    wait_edata(i + 1, 1 - B)
                    issue_gather(1 - B, 0, 1 - p)
                    load_edata(i + 2, B)
                issue_scatter(B, j, p)

        body(0, 0, first=True)

        def pair(q, carry):
            body(2 * q + 1, 1)
            body(2 * q + 2, 0)
            return carry

        lax.fori_loop(0, (NSUPER - 1) // 2, pair, 0)

        lastB = (NSUPER - 1) % 2
        p4 = (lastB + SUP - 1) % 2
        wait_scatter(lastB, SUP - 2, 1 - p4)
        wait_scatter(lastB, SUP - 1, p4)
        wait_gather(1 - lastB, 0, 1 - p4)
        wait_edata(NSUPER, lastB)
        plsc.subcore_barrier()

        def wb_chunk(i, carry):
            j = s + NSUB * i
            base = pl.multiple_of(j * ROWBLK, 8)
            obase = pl.multiple_of(chalf + j * ROWBLK, 8)
            pltpu.sync_copy(acc.at[pl.ds(base, ROWBLK)],
                            out_hbm.at[pl.ds(obase, ROWBLK)])
            return carry

        nz2 = (NROWBLK - s + NSUB - 1) // NSUB
        lax.fori_loop(0, nz2, wb_chunk, 0)

    return layer


def _mean4(e0, e1, e2, e3):
    def body(a, b, c, d, o):
        o[...] = (a[...] + b[...] + c[...] + d[...]) * 0.25

    blk = pl.BlockSpec((1000, EMB_DIM), lambda i: (i, 0))
    return pl.pallas_call(
        body,
        grid=(N_NODES // 1000,),
        in_specs=[blk] * 4,
        out_specs=blk,
        out_shape=jax.ShapeDtypeStruct((N_NODES, EMB_DIM), jnp.float32),
    )(e0, e1, e2, e3)


def kernel(edge_index, adj_values, emb_user, emb_item):
    src = edge_index[0].astype(jnp.int32)
    dst = edge_index[1].astype(jnp.int32)
    w = adj_values.astype(jnp.float32)
    e0 = jnp.concatenate([emb_user, emb_item], axis=0)

    edata = jnp.stack(
        [src.reshape(NCHUNKS, CH), dst.reshape(NCHUNKS, CH)], axis=1)
    edata = (edata.reshape(NSUPER * SUP, NSUB, 2, CH)
             .transpose(1, 0, 2, 3)
             .reshape(NSUB, NSUPER, SUP, 2, CH)
             .transpose(0, 1, 3, 2, 4)
             .reshape(NSUB, NSUPER, 2, SCH))
    wdata = (w.reshape(NSUPER * SUP, NSUB, CH)
             .transpose(1, 0, 2)
             .reshape(NSUB, NSUPER, SCH))

    layer = _layer_kernel()
    e1 = layer(e0, edata, wdata)
    e2 = layer(e1, edata, wdata)
    e3 = layer(e2, edata, wdata)

    final = _mean4(e0, e1, e2, e3)
    stack = jnp.stack([e0, e1, e2, e3], axis=1)
    return final[:NUM_USERS], final[NUM_USERS:], stack


# speedup vs baseline: 1.6512x; 1.6512x over previous
import functools

import jax
import jax.numpy as jnp
from jax import lax
from jax.experimental import pallas as pl
from jax.experimental.pallas import tpu as pltpu
from jax.experimental.pallas import tpu_sc as plsc

NUM_USERS = 25000
NUM_ITEMS = 25000
N_NODES = NUM_USERS + NUM_ITEMS
EMB_DIM = 64
N_EDGES = 800000
N_LAYERS = 3

HALF = N_NODES // 2
CH = 80
SUP = 5
SCH = SUP * CH
NCHUNKS = N_EDGES // CH
NSUB = 16
NSUPER = NCHUNKS // (NSUB * SUP)
ROWBLK = 40
NROWBLK = HALF // ROWBLK


def _layer_kernel():
    mesh = plsc.VectorSubcoreMesh(core_axis_name="c", subcore_axis_name="s",
                                  num_cores=2, num_subcores=NSUB)

    @functools.partial(
        pl.kernel,
        out_type=jax.ShapeDtypeStruct((N_NODES, EMB_DIM), jnp.float32),
        mesh=mesh,
        compiler_params=pltpu.CompilerParams(use_tc_tiling_on_sc=False),
        scratch_types=[
            pltpu.VMEM((2, 2, SCH), jnp.int32),
            pltpu.VMEM((2, SCH), jnp.float32),
            pltpu.VMEM((2, SUP, CH), jnp.int32),
            pltpu.VMEM((2, SUP, CH), jnp.float32),
            pltpu.VMEM((2, CH, EMB_DIM), jnp.float32),
            pltpu.VMEM((2, CH, EMB_DIM), jnp.float32),
            pltpu.VMEM((ROWBLK, EMB_DIM), jnp.float32),
            pltpu.VMEM_SHARED((HALF, EMB_DIM), jnp.float32),
            pltpu.SemaphoreType.DMA,
            pltpu.SemaphoreType.DMA,
            pltpu.SemaphoreType.DMA,
            pltpu.SemaphoreType.DMA,
            pltpu.SemaphoreType.DMA,
            pltpu.SemaphoreType.DMA,
        ],
    )
    def layer(table_hbm, edata_hbm, wdata_hbm, out_hbm,
              edv, wv, dl, wb, rows, rows2, zbuf, acc,
              se0, se1, sg0, sg1, ss0, ss1):
        c = lax.axis_index("c")
        s = lax.axis_index("s")
        chalf = c * HALF
        sem_e = (se0, se1)
        sem_g = (sg0, sg1)
        sem_s = (ss0, ss1)

        def sup_id(i):
            return jnp.minimum(i, NSUPER - 1)

        def load_edata(i, B):
            pltpu.async_copy(edata_hbm.at[s, sup_id(i)], edv.at[B], sem_e[B])
            pltpu.async_copy(wdata_hbm.at[s, sup_id(i)], wv.at[B], sem_e[B])

        def wait_edata(i, B):
            pltpu.make_async_copy(edata_hbm.at[s, sup_id(i)], edv.at[B],
                                  sem_e[B]).wait()
            pltpu.make_async_copy(wdata_hbm.at[s, sup_id(i)], wv.at[B],
                                  sem_e[B]).wait()

        def issue_gather(B, j, p):
            pltpu.async_copy(table_hbm.at[edv.at[B, 0, pl.ds(j * CH, CH)]],
                             rows.at[p], sem_g[p])

        def wait_gather(B, j, p):
            pltpu.make_async_copy(table_hbm.at[edv.at[B, 0, pl.ds(j * CH, CH)]],
                                  rows.at[p], sem_g[p]).wait()

        def issue_scatter(B, j, p):
            pltpu.async_copy(rows2.at[p], acc.at[dl.at[B, j]], sem_s[p],
                             add=True)

        def wait_scatter(B, j, p):
            pltpu.make_async_copy(rows2.at[p], acc.at[dl.at[B, j]],
                                  sem_s[p]).wait()

        def dfold(B):
            for v in range(SCH // 16):
                j, g = v // (CH // 16), v % (CH // 16)
                sl = pl.ds(v * 16, 16)
                d = edv[B, 1, sl]
                w = wv[B, sl]
                fold = jnp.where(d >= HALF, d - HALF, d)
                valid = (d >= chalf) & (d < chalf + HALF)
                dl[B, j, pl.ds(g * 16, 16)] = fold
                wb[B, j, pl.ds(g * 16, 16)] = jnp.where(valid, w, 0.0)

        def scale(B, j, p):
            def scale_group(g, carry):
                ev = wb[B, j, pl.ds(g * 16, 16)]
                for l in range(16):
                    e = g * 16 + l
                    wbc = jnp.broadcast_to(
                        lax.squeeze(lax.slice(ev, (l,), (l + 1,)), (0,)),
                        (16,))
                    for q in range(EMB_DIM // 16):
                        qs = pl.ds(q * 16, 16)
                        rows2[p, e, qs] = rows[p, e, qs] * wbc
                return carry

            lax.fori_loop(0, CH // 16, scale_group, 0)

        load_edata(0, 0)
        wait_edata(0, 0)
        issue_gather(0, 0, 0)
        load_edata(1, 1)

        zeros16 = jnp.zeros((16,), jnp.float32)

        def zb(t, carry):
            r = t // (EMB_DIM // 16)
            k = t % (EMB_DIM // 16)
            zbuf[r, pl.ds(k * 16, 16)] = zeros16
            return carry

        lax.fori_loop(0, ROWBLK * (EMB_DIM // 16), zb, 0)

        def zero_chunk(i, carry):
            j = s + NSUB * i
            base = pl.multiple_of(j * ROWBLK, 8)
            pltpu.sync_copy(zbuf, acc.at[pl.ds(base, ROWBLK)])
            return carry

        nz = (NROWBLK - s + NSUB - 1) // NSUB
        lax.fori_loop(0, nz, zero_chunk, 0)
        plsc.subcore_barrier()

        def body(i, B, first=False):
            dfold(B)
            for j in range(SUP):
                p = (B + j) % 2
                if j < SUP - 1:
                    issue_gather(B, j + 1, 1 - p)
                else:
                    wait_edata(i + 1, 1 - B)
                    issue_gather(1 - B, 0, 1 - p)
                    load_edata(i + 2, B)
                wait_gather(B, j, p)
                if not (first and j < 2):
                    wait_scatter(B, j, p)
                scale(B, j, p)
                issue_scatter(B, j, p)

        body(0, 0, first=True)

        def pair(q, carry):
            body(2 * q + 1, 1)
            body(2 * q + 2, 0)
            return carry

        lax.fori_loop(0, (NSUPER - 1) // 2, pair, 0)

        lastB = (NSUPER - 1) % 2
        p4 = (lastB + SUP - 1) % 2
        wait_scatter(lastB, SUP - 2, 1 - p4)
        wait_scatter(lastB, SUP - 1, p4)
        wait_gather(1 - lastB, 0, 1 - p4)
        wait_edata(NSUPER, lastB)
        plsc.subcore_barrier()

        def wb_chunk(i, carry):
            j = s + NSUB * i
            base = pl.multiple_of(j * ROWBLK, 8)
            obase = pl.multiple_of(chalf + j * ROWBLK, 8)
            pltpu.sync_copy(acc.at[pl.ds(base, ROWBLK)],
                            out_hbm.at[pl.ds(obase, ROWBLK)])
            return carry

        nz2 = (NROWBLK - s + NSUB - 1) // NSUB
        lax.fori_loop(0, nz2, wb_chunk, 0)

    return layer


def _mean4(e0, e1, e2, e3):
    def body(a, b, c, d, o):
        o[...] = (a[...] + b[...] + c[...] + d[...]) * 0.25

    blk = pl.BlockSpec((1000, EMB_DIM), lambda i: (i, 0))
    return pl.pallas_call(
        body,
        grid=(N_NODES // 1000,),
        in_specs=[blk] * 4,
        out_specs=blk,
        out_shape=jax.ShapeDtypeStruct((N_NODES, EMB_DIM), jnp.float32),
    )(e0, e1, e2, e3)


def kernel(edge_index, adj_values, emb_user, emb_item):
    src = edge_index[0].astype(jnp.int32)
    dst = edge_index[1].astype(jnp.int32)
    w = adj_values.astype(jnp.float32)
    e0 = jnp.concatenate([emb_user, emb_item], axis=0)

    edata = jnp.stack(
        [src.reshape(NCHUNKS, CH), dst.reshape(NCHUNKS, CH)], axis=1)
    edata = (edata.reshape(NSUPER * SUP, NSUB, 2, CH)
             .transpose(1, 0, 2, 3)
             .reshape(NSUB, NSUPER, SUP, 2, CH)
             .transpose(0, 1, 3, 2, 4)
             .reshape(NSUB, NSUPER, 2, SCH))
    wdata = (w.reshape(NSUPER * SUP, NSUB, CH)
             .transpose(1, 0, 2)
             .reshape(NSUB, NSUPER, SCH))

    layer = _layer_kernel()
    e1 = layer(e0, edata, wdata)
    e2 = layer(e1, edata, wdata)
    e3 = layer(e2, edata, wdata)

    final = _mean4(e0, e1, e2, e3)
    stack = jnp.stack([e0, e1, e2, e3], axis=1)
    return final[:NUM_USERS], final[NUM_USERS:], stack
